# Initial kernel scaffold; baseline (speedup 1.0000x reference)
import jax
import jax.numpy as jnp
from jax.experimental import pallas as pl


def kernel(x, edge_index, edge_weight, W, alpha):
    raise NotImplementedError("write your pallas kernel here")


# speedup vs baseline: 5.3208x; 5.3208x over previous
import functools

import jax
import jax.numpy as jnp
from jax import lax
from jax.experimental import pallas as pl
from jax.experimental.pallas import tpu as pltpu
from jax.experimental.pallas import tpu_sc as plsc

NC = 2
NS = 16
LANES = 16
NW = NC * NS
C = 128


def _matmul_body(x_ref, w_ref, o_ref):
    o_ref[...] = jnp.dot(x_ref[...], w_ref[...],
                         preferred_element_type=jnp.float32)


def _combine_body(p_ref, a_ref, o_ref):
    a = a_ref[0, 0]
    o_ref[...] = jnp.maximum(a * (p_ref[0] + p_ref[1]), 0.0)


@functools.cache
def _make_sc_spmm(N, H, E_pad):
    CPW = E_pad // (NW * C)
    RPT = N // NS
    nfull = RPT // C
    rem = RPT % C
    mesh = plsc.VectorSubcoreMesh(core_axis_name="c", subcore_axis_name="s")

    @functools.partial(
        pl.kernel,
        out_type=jax.ShapeDtypeStruct((NC * N, H), jnp.float32),
        mesh=mesh,
        compiler_params=pltpu.CompilerParams(use_tc_tiling_on_sc=False),
        scratch_types=[
            pltpu.VMEM((C,), jnp.int32),
            pltpu.VMEM((C,), jnp.int32),
            pltpu.VMEM((C,), jnp.float32),
            pltpu.VMEM((C, H), jnp.float32),
            pltpu.VMEM((C, H), jnp.float32),
            pltpu.VMEM_SHARED((N, H), jnp.float32),
            pltpu.SemaphoreType.DMA,
        ],
    )
    def sc_spmm(y_hbm, src_hbm, dst_hbm, w_hbm, out_hbm,
                src_v, dst_v, w_v, rows_v, msgs_v, acc_sh, sem):
        cid = lax.axis_index("c")
        sid = lax.axis_index("s")
        wid = cid * NS + sid

        zero = jnp.zeros((LANES,), jnp.float32)
        for r in range(C):
            for k in range(H // LANES):
                msgs_v[r, pl.ds(k * LANES, LANES)] = zero
        r0 = sid * RPT
        for j in range(nfull):
            pltpu.sync_copy(msgs_v, acc_sh.at[pl.ds(r0 + j * C, C)])
        if rem:
            pltpu.sync_copy(msgs_v.at[pl.ds(0, rem)],
                            acc_sh.at[pl.ds(r0 + nfull * C, rem)])
        plsc.subcore_barrier()

        ebase = wid * (CPW * C)

        @pl.loop(0, CPW)
        def _chunk(k):
            e0 = ebase + k * C
            pltpu.sync_copy(src_hbm.at[pl.ds(e0, C)], src_v)
            pltpu.sync_copy(dst_hbm.at[pl.ds(e0, C)], dst_v)
            pltpu.sync_copy(w_hbm.at[pl.ds(e0, C)], w_v)
            pltpu.async_copy(y_hbm.at[src_v], rows_v, sem).wait()
            dnums = lax.GatherDimensionNumbers(
                offset_dims=(), collapsed_slice_dims=(0,), start_index_map=(0,))
            for g in range(C // LANES):
                w_vec = w_v[pl.ds(g * LANES, LANES)]
                for j in range(LANES):
                    idx = jnp.full((LANES, 1), j, jnp.int32)
                    wsp = lax.gather(
                        w_vec, idx, dnums, (1,),
                        mode=lax.GatherScatterMode.PROMISE_IN_BOUNDS)
                    e = g * LANES + j
                    for k4 in range(H // LANES):
                        sl = pl.ds(k4 * LANES, LANES)
                        msgs_v[e, sl] = rows_v[e, sl] * wsp
            pltpu.sync_copy(msgs_v, acc_sh.at[dst_v], add=True)

        plsc.subcore_barrier()

        ob = cid * N + r0
        for j in range(nfull):
            pltpu.sync_copy(acc_sh.at[pl.ds(r0 + j * C, C)],
                            out_hbm.at[pl.ds(ob + j * C, C)])
        if rem:
            pltpu.sync_copy(acc_sh.at[pl.ds(r0 + nfull * C, rem)],
                            out_hbm.at[pl.ds(ob + nfull * C, rem)])

    return sc_spmm


def kernel(x, edge_index, edge_weight, W, alpha):
    N, D = x.shape
    H = W.shape[1]
    E = edge_weight.shape[0]

    y = pl.pallas_call(
        _matmul_body,
        out_shape=jax.ShapeDtypeStruct((N, H), jnp.float32),
    )(x, W)

    grain = NW * C
    E_pad = ((E + grain - 1) // grain) * grain
    pad = E_pad - E
    src = jnp.pad(edge_index[0], (0, pad))
    dst = jnp.pad(edge_index[1], (0, pad))
    w = jnp.pad(edge_weight, (0, pad))

    rgrain = NS * C
    N_pad = ((N + rgrain - 1) // rgrain) * rgrain
    partials = _make_sc_spmm(N_pad, H, E_pad)(y, src, dst, w)
    p = partials.reshape(NC, N_pad, H)[:, :N, :]

    out = pl.pallas_call(
        _combine_body,
        out_shape=jax.ShapeDtypeStruct((N, H), jnp.float32),
    )(p, alpha.reshape(1, 1))
    return out
